# baseline (speedup 1.0000x reference)
import functools

import jax
import jax.numpy as jnp
from jax import lax
from jax.experimental import pallas as pl
from jax.experimental.pallas import tpu as pltpu
from jax.experimental.pallas import tpu_sc as plsc

N = 10000
E = 320000
F = 128
C = 64
NC = 2
NS = 16
NW = NC * NS
EPW = E // NW
CH = 80
NCHUNK = EPW // CH
NPAD = N
RPS = N // NS
RCH = 125
DEGW = 16

_mesh = plsc.VectorSubcoreMesh(core_axis_name="c", subcore_axis_name="s")
_sc_params = pltpu.CompilerParams(use_tc_tiling_on_sc=False)


def _fill_f32(ref, rows, cols, value):
    v = jnp.full((16,), value, jnp.float32)
    per_row = cols // 16

    @pl.loop(0, rows * per_row)
    def _(k):
        i = k // per_row
        j = k % per_row
        ref[i, pl.ds(j * 16, 16)] = v


def _deg_body(ei4_hbm, degp_hbm, didx, ones_v, stage_v, acc):
    cid = lax.axis_index("c")
    sid = lax.axis_index("s")
    wid = sid * NC + cid

    _fill_f32(stage_v, RCH, DEGW, 0.0)
    for t in range(RPS // RCH):
        pltpu.sync_copy(stage_v, acc.at[pl.ds((sid * (RPS // RCH) + t) * RCH, RCH)])
    _fill_f32(ones_v, CH, DEGW, 1.0)
    pltpu.sync_copy(ei4_hbm.at[1, wid], didx)
    plsc.subcore_barrier()

    @pl.loop(0, NCHUNK)
    def _(j):
        pltpu.sync_copy(ones_v, acc.at[didx.at[j]], add=True)

    plsc.subcore_barrier()
    for t in range(RPS // RCH):
        r0 = (sid * (RPS // RCH) + t) * RCH
        pltpu.sync_copy(acc.at[pl.ds(r0, RCH)], stage_v)
        pltpu.sync_copy(stage_v, degp_hbm.at[cid, pl.ds(r0, RCH)])


_deg_call = functools.partial(
    pl.kernel,
    out_type=jax.ShapeDtypeStruct((NC, N, DEGW), jnp.float32),
    mesh=_mesh,
    scratch_types=[
        pltpu.VMEM((NCHUNK, CH), jnp.int32),
        pltpu.VMEM((CH, DEGW), jnp.float32),
        pltpu.VMEM((RCH, DEGW), jnp.float32),
        pltpu.VMEM_SHARED((NPAD, DEGW), jnp.float32),
    ],
    compiler_params=_sc_params,
)(_deg_body)


def _agg_body(ei4_hbm, hp_hbm, aggp_hbm, sidx, didx, rows0, rows1, stage_v, sem0, sem1, acc):
    cid = lax.axis_index("c")
    sid = lax.axis_index("s")
    wid = sid * NC + cid

    _fill_f32(stage_v, RCH, C, 0.0)
    for t in range(RPS // RCH):
        pltpu.sync_copy(stage_v, acc.at[pl.ds((sid * (RPS // RCH) + t) * RCH, RCH)])
    pltpu.sync_copy(ei4_hbm.at[0, wid], sidx)
    pltpu.sync_copy(ei4_hbm.at[1, wid], didx)
    plsc.subcore_barrier()

    pltpu.async_copy(hp_hbm.at[sidx.at[0]], rows0, sem0)

    @pl.loop(0, NCHUNK // 2)
    def _(p):
        j0 = 2 * p
        pltpu.make_async_copy(hp_hbm.at[sidx.at[j0]], rows0, sem0).wait()
        pltpu.async_copy(hp_hbm.at[sidx.at[j0 + 1]], rows1, sem1)
        pltpu.sync_copy(rows0, acc.at[didx.at[j0]], add=True)
        pltpu.make_async_copy(hp_hbm.at[sidx.at[j0 + 1]], rows1, sem1).wait()

        @pl.when(p + 1 < NCHUNK // 2)
        def _():
            pltpu.async_copy(hp_hbm.at[sidx.at[j0 + 2]], rows0, sem0)

        pltpu.sync_copy(rows1, acc.at[didx.at[j0 + 1]], add=True)

    if NCHUNK % 2 == 1:
        pltpu.sync_copy(hp_hbm.at[sidx.at[NCHUNK - 1]], rows0)
        pltpu.sync_copy(rows0, acc.at[didx.at[NCHUNK - 1]], add=True)

    plsc.subcore_barrier()
    for t in range(RPS // RCH):
        r0 = (sid * (RPS // RCH) + t) * RCH
        pltpu.sync_copy(acc.at[pl.ds(r0, RCH)], stage_v)
        pltpu.sync_copy(stage_v, aggp_hbm.at[cid, pl.ds(r0, RCH)])


_agg_call = functools.partial(
    pl.kernel,
    out_type=jax.ShapeDtypeStruct((NC, N, C), jnp.float32),
    mesh=_mesh,
    scratch_types=[
        pltpu.VMEM((NCHUNK, CH), jnp.int32),
        pltpu.VMEM((NCHUNK, CH), jnp.int32),
        pltpu.VMEM((CH, C), jnp.float32),
        pltpu.VMEM((CH, C), jnp.float32),
        pltpu.VMEM((RCH, C), jnp.float32),
        pltpu.SemaphoreType.DMA,
        pltpu.SemaphoreType.DMA,
        pltpu.VMEM_SHARED((NPAD, C), jnp.float32),
    ],
    compiler_params=_sc_params,
)(_agg_body)


BLK = 2000


def _mm_body(x_ref, w_ref, degp_ref, hp_ref):
    d = degp_ref[...]
    deg = 1.0 + d[0, :, 0:1] + d[1, :, 0:1]
    dinv = lax.rsqrt(deg)
    h = jnp.dot(x_ref[...], w_ref[...], preferred_element_type=jnp.float32)
    hp_ref[...] = h * dinv


_mm_call = pl.pallas_call(
    _mm_body,
    grid=(N // BLK,),
    in_specs=[
        pl.BlockSpec((BLK, F), lambda i: (i, 0)),
        pl.BlockSpec((F, C), lambda i: (0, 0)),
        pl.BlockSpec((NC, BLK, DEGW), lambda i: (0, i, 0)),
    ],
    out_specs=pl.BlockSpec((BLK, C), lambda i: (i, 0)),
    out_shape=jax.ShapeDtypeStruct((N, C), jnp.float32),
)


def _fin_body(aggp_ref, hp_ref, degp_ref, b_ref, o_ref):
    d = degp_ref[...]
    deg = 1.0 + d[0, :, 0:1] + d[1, :, 0:1]
    dinv = lax.rsqrt(deg)
    a = aggp_ref[...]
    pre = dinv * (a[0] + a[1] + hp_ref[...]) + b_ref[...]
    m = jnp.max(pre, axis=1, keepdims=True)
    z = pre - m
    lse = jnp.log(jnp.sum(jnp.exp(z), axis=1, keepdims=True))
    o_ref[...] = z - lse


_fin_call = pl.pallas_call(
    _fin_body,
    grid=(N // BLK,),
    in_specs=[
        pl.BlockSpec((NC, BLK, C), lambda i: (0, i, 0)),
        pl.BlockSpec((BLK, C), lambda i: (i, 0)),
        pl.BlockSpec((NC, BLK, DEGW), lambda i: (0, i, 0)),
        pl.BlockSpec((1, C), lambda i: (0, 0)),
    ],
    out_specs=pl.BlockSpec((BLK, C), lambda i: (i, 0)),
    out_shape=jax.ShapeDtypeStruct((N, C), jnp.float32),
)


def kernel(x, edge_index, W, b):
    ei4 = edge_index.reshape(2, NW, NCHUNK, CH)
    degp = _deg_call(ei4)
    hp = _mm_call(x, W, degp)
    aggp = _agg_call(ei4, hp)
    return _fin_call(aggp, hp, degp, b.reshape(1, C))


# speedup vs baseline: 1.2636x; 1.2636x over previous
import functools

import jax
import jax.numpy as jnp
from jax import lax
from jax.experimental import pallas as pl
from jax.experimental.pallas import tpu as pltpu
from jax.experimental.pallas import tpu_sc as plsc

N = 10000
E = 320000
F = 128
C = 64
NC = 2
NS = 16
NW = NC * NS
EPW = E // NW
CH = 80
NCHUNK = EPW // CH
NPAD = N
RPS = N // NS
RCH = 125
DEGW = 16

_mesh = plsc.VectorSubcoreMesh(core_axis_name="c", subcore_axis_name="s")
_sc_params = pltpu.CompilerParams(use_tc_tiling_on_sc=False)


def _fill_f32(ref, rows, cols, value):
    v = jnp.full((16,), value, jnp.float32)
    per_row = cols // 16

    @pl.loop(0, rows * per_row)
    def _(k):
        i = k // per_row
        j = k % per_row
        ref[i, pl.ds(j * 16, 16)] = v


def _deg_body(ei4_hbm, degp_hbm, didx, ones_v, stage_v, acc):
    cid = lax.axis_index("c")
    sid = lax.axis_index("s")
    wid = sid * NC + cid

    _fill_f32(stage_v, RCH, DEGW, 0.0)
    for t in range(RPS // RCH):
        pltpu.sync_copy(stage_v, acc.at[pl.ds((sid * (RPS // RCH) + t) * RCH, RCH)])
    _fill_f32(ones_v, CH, DEGW, 1.0)
    pltpu.sync_copy(ei4_hbm.at[1, wid], didx)
    plsc.subcore_barrier()

    @pl.loop(0, NCHUNK)
    def _(j):
        pltpu.sync_copy(ones_v, acc.at[didx.at[j]], add=True)

    plsc.subcore_barrier()
    for t in range(RPS // RCH):
        r0 = (sid * (RPS // RCH) + t) * RCH
        pltpu.sync_copy(acc.at[pl.ds(r0, RCH)], stage_v)
        pltpu.sync_copy(stage_v, degp_hbm.at[cid, pl.ds(r0, RCH)])


_deg_call = functools.partial(
    pl.kernel,
    out_type=jax.ShapeDtypeStruct((NC, N, DEGW), jnp.float32),
    mesh=_mesh,
    scratch_types=[
        pltpu.VMEM((NCHUNK, CH), jnp.int32),
        pltpu.VMEM((CH, DEGW), jnp.float32),
        pltpu.VMEM((RCH, DEGW), jnp.float32),
        pltpu.VMEM_SHARED((NPAD, DEGW), jnp.float32),
    ],
    compiler_params=_sc_params,
)(_deg_body)


def _agg_body(ei4_hbm, hp_hbm, aggp_hbm, sidx, didx,
              rows0, rows1, rows2, rows3, stage_v,
              g0, g1, g2, g3, s0, s1, s2, s3, acc):
    cid = lax.axis_index("c")
    sid = lax.axis_index("s")
    wid = sid * NC + cid
    rows = (rows0, rows1, rows2, rows3)
    gsem = (g0, g1, g2, g3)
    ssem = (s0, s1, s2, s3)
    NQ = NCHUNK // 4

    def fire_g(j, b):
        pltpu.async_copy(hp_hbm.at[sidx.at[j]], rows[b], gsem[b])

    def wait_g(j, b):
        pltpu.make_async_copy(hp_hbm.at[sidx.at[j]], rows[b], gsem[b]).wait()

    def fire_s(j, b):
        pltpu.async_copy(rows[b], acc.at[didx.at[j]], ssem[b], add=True)

    def wait_s(j, b):
        pltpu.make_async_copy(rows[b], acc.at[didx.at[j]], ssem[b]).wait()

    _fill_f32(stage_v, RCH, C, 0.0)
    for t in range(RPS // RCH):
        pltpu.sync_copy(stage_v, acc.at[pl.ds((sid * (RPS // RCH) + t) * RCH, RCH)])
    pltpu.sync_copy(ei4_hbm.at[0, wid], sidx)
    pltpu.sync_copy(ei4_hbm.at[1, wid], didx)
    plsc.subcore_barrier()

    fire_g(0, 0)
    fire_g(1, 1)
    for j in range(4):
        wait_g(j, j)
        fire_s(j, j)
        if j >= 2:
            wait_s(j - 2, j - 2)
        fire_g(j + 2, (j + 2) % 4)

    @pl.loop(1, NQ)
    def _(q):
        j0 = 4 * q
        for b in range(4):
            j = j0 + b
            wait_g(j, b)
            fire_s(j, b)
            wait_s(j - 2, (b + 2) % 4)

            @pl.when(j + 2 < 4 * NQ)
            def _():
                fire_g(j + 2, (b + 2) % 4)

    for j in range(4 * NQ, NCHUNK):
        pltpu.sync_copy(hp_hbm.at[sidx.at[j]], rows0)
        pltpu.sync_copy(rows0, acc.at[didx.at[j]], add=True)
    wait_s(4 * NQ - 2, 2)
    wait_s(4 * NQ - 1, 3)

    plsc.subcore_barrier()
    for t in range(RPS // RCH):
        r0 = (sid * (RPS // RCH) + t) * RCH
        pltpu.sync_copy(acc.at[pl.ds(r0, RCH)], stage_v)
        pltpu.sync_copy(stage_v, aggp_hbm.at[cid, pl.ds(r0, RCH)])


_agg_call = functools.partial(
    pl.kernel,
    out_type=jax.ShapeDtypeStruct((NC, N, C), jnp.float32),
    mesh=_mesh,
    scratch_types=[
        pltpu.VMEM((NCHUNK, CH), jnp.int32),
        pltpu.VMEM((NCHUNK, CH), jnp.int32),
        pltpu.VMEM((CH, C), jnp.float32),
        pltpu.VMEM((CH, C), jnp.float32),
        pltpu.VMEM((CH, C), jnp.float32),
        pltpu.VMEM((CH, C), jnp.float32),
        pltpu.VMEM((RCH, C), jnp.float32),
        pltpu.SemaphoreType.DMA,
        pltpu.SemaphoreType.DMA,
        pltpu.SemaphoreType.DMA,
        pltpu.SemaphoreType.DMA,
        pltpu.SemaphoreType.DMA,
        pltpu.SemaphoreType.DMA,
        pltpu.SemaphoreType.DMA,
        pltpu.SemaphoreType.DMA,
        pltpu.VMEM_SHARED((NPAD, C), jnp.float32),
    ],
    compiler_params=_sc_params,
)(_agg_body)


BLK = 2000


def _mm_body(x_ref, w_ref, degp_ref, hp_ref):
    d = degp_ref[...]
    deg = 1.0 + d[0, :, 0:1] + d[1, :, 0:1]
    dinv = lax.rsqrt(deg)
    h = jnp.dot(x_ref[...], w_ref[...], preferred_element_type=jnp.float32)
    hp_ref[...] = h * dinv


_mm_call = pl.pallas_call(
    _mm_body,
    grid=(N // BLK,),
    in_specs=[
        pl.BlockSpec((BLK, F), lambda i: (i, 0)),
        pl.BlockSpec((F, C), lambda i: (0, 0)),
        pl.BlockSpec((NC, BLK, DEGW), lambda i: (0, i, 0)),
    ],
    out_specs=pl.BlockSpec((BLK, C), lambda i: (i, 0)),
    out_shape=jax.ShapeDtypeStruct((N, C), jnp.float32),
)


def _fin_body(aggp_ref, hp_ref, degp_ref, b_ref, o_ref):
    d = degp_ref[...]
    deg = 1.0 + d[0, :, 0:1] + d[1, :, 0:1]
    dinv = lax.rsqrt(deg)
    a = aggp_ref[...]
    pre = dinv * (a[0] + a[1] + hp_ref[...]) + b_ref[...]
    m = jnp.max(pre, axis=1, keepdims=True)
    z = pre - m
    lse = jnp.log(jnp.sum(jnp.exp(z), axis=1, keepdims=True))
    o_ref[...] = z - lse


_fin_call = pl.pallas_call(
    _fin_body,
    grid=(N // BLK,),
    in_specs=[
        pl.BlockSpec((NC, BLK, C), lambda i: (0, i, 0)),
        pl.BlockSpec((BLK, C), lambda i: (i, 0)),
        pl.BlockSpec((NC, BLK, DEGW), lambda i: (0, i, 0)),
        pl.BlockSpec((1, C), lambda i: (0, 0)),
    ],
    out_specs=pl.BlockSpec((BLK, C), lambda i: (i, 0)),
    out_shape=jax.ShapeDtypeStruct((N, C), jnp.float32),
)


def kernel(x, edge_index, W, b):
    ei4 = edge_index.reshape(2, NW, NCHUNK, CH)
    degp = _deg_call(ei4)
    hp = _mm_call(x, W, degp)
    aggp = _agg_call(ei4, hp)
    return _fin_call(aggp, hp, degp, b.reshape(1, C))


# speedup vs baseline: 1.3068x; 1.0342x over previous
import functools

import jax
import jax.numpy as jnp
from jax import lax
from jax.experimental import pallas as pl
from jax.experimental.pallas import tpu as pltpu
from jax.experimental.pallas import tpu_sc as plsc

N = 10000
E = 320000
F = 128
C = 64
NC = 2
NS = 16
NW = NC * NS
EPW = E // NW
CH = 80
NCHUNK = EPW // CH
NPAD = N
RPS = N // NS
RCH = 125
DEGW = 16

_mesh = plsc.VectorSubcoreMesh(core_axis_name="c", subcore_axis_name="s")
_sc_params = pltpu.CompilerParams(use_tc_tiling_on_sc=False)


def _fill_f32(ref, rows, cols, value):
    v = jnp.full((16,), value, jnp.float32)
    per_row = cols // 16

    @pl.loop(0, rows * per_row)
    def _(k):
        i = k // per_row
        j = k % per_row
        ref[i, pl.ds(j * 16, 16)] = v


def _deg_body(ei4_hbm, degp_hbm, didx, ones_v, stage_v, d0, d1, d2, d3, acc):
    cid = lax.axis_index("c")
    sid = lax.axis_index("s")
    wid = sid * NC + cid

    _fill_f32(stage_v, RCH, DEGW, 0.0)
    for t in range(RPS // RCH):
        pltpu.sync_copy(stage_v, acc.at[pl.ds((sid * (RPS // RCH) + t) * RCH, RCH)])
    _fill_f32(ones_v, CH, DEGW, 1.0)
    pltpu.sync_copy(ei4_hbm.at[1, wid], didx)
    plsc.subcore_barrier()

    ssem = (d0, d1, d2, d3)
    NQ4 = NCHUNK // 4

    def fire_s(j, b):
        pltpu.async_copy(ones_v, acc.at[didx.at[j]], ssem[b], add=True)

    def wait_s(j, b):
        pltpu.make_async_copy(ones_v, acc.at[didx.at[j]], ssem[b]).wait()

    for b in range(4):
        fire_s(b, b)

    @pl.loop(1, NQ4)
    def _(q):
        j0 = 4 * q
        for b in range(4):
            wait_s(j0 + b - 4, b)
            fire_s(j0 + b, b)

    for b in range(4):
        wait_s(4 * NQ4 - 4 + b, b)
    for j in range(4 * NQ4, NCHUNK):
        pltpu.sync_copy(ones_v, acc.at[didx.at[j]], add=True)

    plsc.subcore_barrier()
    for t in range(RPS // RCH):
        r0 = (sid * (RPS // RCH) + t) * RCH
        pltpu.sync_copy(acc.at[pl.ds(r0, RCH)], stage_v)
        pltpu.sync_copy(stage_v, degp_hbm.at[cid, pl.ds(r0, RCH)])


_deg_call = functools.partial(
    pl.kernel,
    out_type=jax.ShapeDtypeStruct((NC, N, DEGW), jnp.float32),
    mesh=_mesh,
    scratch_types=[
        pltpu.VMEM((NCHUNK, CH), jnp.int32),
        pltpu.VMEM((CH, DEGW), jnp.float32),
        pltpu.VMEM((RCH, DEGW), jnp.float32),
        pltpu.SemaphoreType.DMA,
        pltpu.SemaphoreType.DMA,
        pltpu.SemaphoreType.DMA,
        pltpu.SemaphoreType.DMA,
        pltpu.VMEM_SHARED((NPAD, DEGW), jnp.float32),
    ],
    compiler_params=_sc_params,
)(_deg_body)


def _agg_body(ei4_hbm, hp_hbm, aggp_hbm, sidx, didx,
              rows0, rows1, rows2, rows3, rows4, stage_v,
              g0, g1, g2, g3, g4, s0, s1, s2, s3, s4, acc):
    cid = lax.axis_index("c")
    sid = lax.axis_index("s")
    wid = sid * NC + cid
    rows = (rows0, rows1, rows2, rows3, rows4)
    gsem = (g0, g1, g2, g3, g4)
    ssem = (s0, s1, s2, s3, s4)
    NQ = NCHUNK // 5

    def fire_g(j, b):
        pltpu.async_copy(hp_hbm.at[sidx.at[j]], rows[b], gsem[b])

    def wait_g(j, b):
        pltpu.make_async_copy(hp_hbm.at[sidx.at[j]], rows[b], gsem[b]).wait()

    def fire_s(j, b):
        pltpu.async_copy(rows[b], acc.at[didx.at[j]], ssem[b], add=True)

    def wait_s(j, b):
        pltpu.make_async_copy(rows[b], acc.at[didx.at[j]], ssem[b]).wait()

    _fill_f32(stage_v, RCH, C, 0.0)
    for t in range(RPS // RCH):
        pltpu.sync_copy(stage_v, acc.at[pl.ds((sid * (RPS // RCH) + t) * RCH, RCH)])
    pltpu.sync_copy(ei4_hbm.at[0, wid], sidx)
    pltpu.sync_copy(ei4_hbm.at[1, wid], didx)
    plsc.subcore_barrier()

    fire_g(0, 0)
    fire_g(1, 1)
    for j in range(5):
        wait_g(j, j)
        fire_s(j, j)
        if j >= 3:
            wait_s(j - 3, j - 3)
        fire_g(j + 2, (j + 2) % 5)

    @pl.loop(1, NQ)
    def _(q):
        j0 = 5 * q
        for b in range(5):
            j = j0 + b
            wait_g(j, b)
            fire_s(j, b)
            wait_s(j - 3, (b + 2) % 5)

            @pl.when(j + 2 < NCHUNK)
            def _():
                fire_g(j + 2, (b + 2) % 5)

    wait_s(NCHUNK - 3, (NCHUNK - 3) % 5)
    wait_s(NCHUNK - 2, (NCHUNK - 2) % 5)
    wait_s(NCHUNK - 1, (NCHUNK - 1) % 5)

    plsc.subcore_barrier()
    for t in range(RPS // RCH):
        r0 = (sid * (RPS // RCH) + t) * RCH
        pltpu.sync_copy(acc.at[pl.ds(r0, RCH)], stage_v)
        pltpu.sync_copy(stage_v, aggp_hbm.at[cid, pl.ds(r0, RCH)])


_agg_call = functools.partial(
    pl.kernel,
    out_type=jax.ShapeDtypeStruct((NC, N, C), jnp.float32),
    mesh=_mesh,
    scratch_types=[
        pltpu.VMEM((NCHUNK, CH), jnp.int32),
        pltpu.VMEM((NCHUNK, CH), jnp.int32),
        pltpu.VMEM((CH, C), jnp.float32),
        pltpu.VMEM((CH, C), jnp.float32),
        pltpu.VMEM((CH, C), jnp.float32),
        pltpu.VMEM((CH, C), jnp.float32),
        pltpu.VMEM((CH, C), jnp.float32),
        pltpu.VMEM((RCH, C), jnp.float32),
        pltpu.SemaphoreType.DMA,
        pltpu.SemaphoreType.DMA,
        pltpu.SemaphoreType.DMA,
        pltpu.SemaphoreType.DMA,
        pltpu.SemaphoreType.DMA,
        pltpu.SemaphoreType.DMA,
        pltpu.SemaphoreType.DMA,
        pltpu.SemaphoreType.DMA,
        pltpu.SemaphoreType.DMA,
        pltpu.SemaphoreType.DMA,
        pltpu.VMEM_SHARED((NPAD, C), jnp.float32),
    ],
    compiler_params=_sc_params,
)(_agg_body)


BLK = 10000


def _mm_body(x_ref, w_ref, degp_ref, hp_ref):
    d = degp_ref[...]
    deg = 1.0 + d[0, :, 0:1] + d[1, :, 0:1]
    dinv = lax.rsqrt(deg)
    h = jnp.dot(x_ref[...], w_ref[...], preferred_element_type=jnp.float32)
    hp_ref[...] = h * dinv


_mm_call = pl.pallas_call(
    _mm_body,
    grid=(N // BLK,),
    in_specs=[
        pl.BlockSpec((BLK, F), lambda i: (i, 0)),
        pl.BlockSpec((F, C), lambda i: (0, 0)),
        pl.BlockSpec((NC, BLK, DEGW), lambda i: (0, i, 0)),
    ],
    out_specs=pl.BlockSpec((BLK, C), lambda i: (i, 0)),
    out_shape=jax.ShapeDtypeStruct((N, C), jnp.float32),
)


def _fin_body(aggp_ref, hp_ref, degp_ref, b_ref, o_ref):
    d = degp_ref[...]
    deg = 1.0 + d[0, :, 0:1] + d[1, :, 0:1]
    dinv = lax.rsqrt(deg)
    a = aggp_ref[...]
    pre = dinv * (a[0] + a[1] + hp_ref[...]) + b_ref[...]
    m = jnp.max(pre, axis=1, keepdims=True)
    z = pre - m
    lse = jnp.log(jnp.sum(jnp.exp(z), axis=1, keepdims=True))
    o_ref[...] = z - lse


_fin_call = pl.pallas_call(
    _fin_body,
    grid=(N // BLK,),
    in_specs=[
        pl.BlockSpec((NC, BLK, C), lambda i: (0, i, 0)),
        pl.BlockSpec((BLK, C), lambda i: (i, 0)),
        pl.BlockSpec((NC, BLK, DEGW), lambda i: (0, i, 0)),
        pl.BlockSpec((1, C), lambda i: (0, 0)),
    ],
    out_specs=pl.BlockSpec((BLK, C), lambda i: (i, 0)),
    out_shape=jax.ShapeDtypeStruct((N, C), jnp.float32),
)


def kernel(x, edge_index, W, b):
    ei4 = edge_index.reshape(2, NW, NCHUNK, CH)
    degp = _deg_call(ei4)
    hp = _mm_call(x, W, degp)
    aggp = _agg_call(ei4, hp)
    return _fin_call(aggp, hp, degp, b.reshape(1, C))
